# initial kernel scaffold (unmeasured)
import jax
import jax.numpy as jnp
from jax import lax
from jax.experimental import pallas as pl
from jax.experimental.pallas import tpu as pltpu

N_DEV = 4
SQ = 256
DH = 128
HQ = 8
HKV = 2
G = HQ // HKV
SCALE = 0.08838834764831843
ROWS = HQ * SQ
LROWS = ROWS // DH
CROWS = ROWS + LROWS


def kernel(x, Wq, Wo, K_ext, V_ext):
    skv = K_ext.shape[1]
    x2 = x.reshape(SQ, HQ * DH)
    K2 = K_ext.reshape(skv, HKV * DH)
    V2 = V_ext.reshape(skv, HKV * DH)

    def body(x_ref, wq_ref, wo_ref, k_ref, v_ref, out_ref,
             comm_ref, send_sems, recv_sems):
        my = lax.axis_index("i")
        left = lax.rem(my + N_DEV - 1, N_DEV)
        right = lax.rem(my + 1, N_DEV)

        barrier_sem = pltpu.get_barrier_semaphore()
        for nbr in (left, right):
            pl.semaphore_signal(
                barrier_sem, inc=1,
                device_id=(nbr,), device_id_type=pl.DeviceIdType.MESH,
            )
        pl.semaphore_wait(barrier_sem, 2)

        xb = x_ref[...].astype(jnp.bfloat16)
        wqb = wq_ref[...].astype(jnp.bfloat16)
        q = jnp.dot(xb, wqb, preferred_element_type=jnp.float32) * SCALE
        qb = q.astype(jnp.bfloat16)

        for g in range(HKV):
            qg = jnp.concatenate(
                [qb[:, (g * G + j) * DH:(g * G + j + 1) * DH] for j in range(G)],
                axis=0,
            )
            kg = k_ref[:, g * DH:(g + 1) * DH].astype(jnp.bfloat16)
            vg = v_ref[:, g * DH:(g + 1) * DH].astype(jnp.bfloat16)
            s = lax.dot_general(qg, kg, (((1,), (1,)), ((), ())),
                                preferred_element_type=jnp.float32)
            p = jnp.exp(s)
            lsum = jnp.sum(p, axis=1)
            og = lax.dot_general(p.astype(jnp.bfloat16), vg,
                                 (((1,), (0,)), ((), ())),
                                 preferred_element_type=jnp.float32)
            comm_ref[0, g * G * SQ:(g + 1) * G * SQ, :] = og
            comm_ref[0, ROWS + g * (LROWS // HKV):
                        ROWS + (g + 1) * (LROWS // HKV), :] = (
                lsum.reshape(LROWS // HKV, DH))

        for h in range(N_DEV - 1):
            rdma = pltpu.make_async_remote_copy(
                src_ref=comm_ref.at[h],
                dst_ref=comm_ref.at[h + 1],
                send_sem=send_sems.at[h],
                recv_sem=recv_sems.at[h],
                device_id=(right,),
                device_id_type=pl.DeviceIdType.MESH,
            )
            rdma.start()
            rdma.wait()

        acc = (comm_ref[0] + comm_ref[1]) + (comm_ref[2] + comm_ref[3])
        lall = acc[ROWS:CROWS, :].reshape(ROWS, 1)
        outn = acc[:ROWS, :] / lall
        attn = jnp.concatenate(
            [outn[h * SQ:(h + 1) * SQ, :] for h in range(HQ)], axis=1)
        out_ref[...] = jnp.dot(attn.astype(jnp.bfloat16),
                               wo_ref[...].astype(jnp.bfloat16),
                               preferred_element_type=jnp.float32)

    out = pl.pallas_call(
        body,
        out_shape=jax.ShapeDtypeStruct((SQ, HQ * DH), jnp.float32),
        in_specs=[pl.BlockSpec(memory_space=pltpu.VMEM)] * 5,
        out_specs=pl.BlockSpec(memory_space=pltpu.VMEM),
        scratch_shapes=[
            pltpu.VMEM((N_DEV, CROWS, DH), jnp.float32),
            pltpu.SemaphoreType.DMA((N_DEV - 1,)),
            pltpu.SemaphoreType.DMA((N_DEV - 1,)),
        ],
        compiler_params=pltpu.CompilerParams(collective_id=0),
    )(x2, Wq, Wo, K2, V2)
    return out.reshape(1, SQ, HQ * DH)


# baseline (device time: 60011 ns/iter reference)
import jax
import jax.numpy as jnp
from jax import lax
from jax.experimental import pallas as pl
from jax.experimental.pallas import tpu as pltpu

N_DEV = 4
SQ = 256
DH = 128
HQ = 8
HKV = 2
G = HQ // HKV
SCALE = 0.08838834764831843
COLS = HQ * SQ
CROWS = DH + 8


def kernel(x, Wq, Wo, K_ext, V_ext):
    skv = K_ext.shape[1]
    x2 = x.reshape(SQ, HQ * DH)
    K2 = K_ext.reshape(skv, HKV * DH)
    V2 = V_ext.reshape(skv, HKV * DH)

    def body(x_ref, wq_ref, wo_ref, k_ref, v_ref, out_ref,
             comm_ref, send_sems, recv_sems):
        my = lax.axis_index("i")
        left = lax.rem(my + N_DEV - 1, N_DEV)
        right = lax.rem(my + 1, N_DEV)

        barrier_sem = pltpu.get_barrier_semaphore()
        for nbr in (left, right):
            pl.semaphore_signal(
                barrier_sem, inc=1,
                device_id=(nbr,), device_id_type=pl.DeviceIdType.MESH,
            )
        pl.semaphore_wait(barrier_sem, 2)

        xb = x_ref[...].astype(jnp.bfloat16)
        wqb = wq_ref[...].astype(jnp.bfloat16)
        q = jnp.dot(xb, wqb, preferred_element_type=jnp.float32) * SCALE
        qb = q.astype(jnp.bfloat16)

        for g in range(HKV):
            qg = jnp.concatenate(
                [qb[:, (g * G + j) * DH:(g * G + j + 1) * DH] for j in range(G)],
                axis=0,
            )
            kg = k_ref[:, g * DH:(g + 1) * DH].astype(jnp.bfloat16)
            vg = v_ref[:, g * DH:(g + 1) * DH].astype(jnp.bfloat16)
            st = lax.dot_general(kg, qg, (((1,), (1,)), ((), ())),
                                 preferred_element_type=jnp.float32)
            pt = jnp.exp(st)
            lsum = jnp.sum(pt, axis=0, keepdims=True)
            ogt = lax.dot_general(vg, pt.astype(jnp.bfloat16),
                                  (((0,), (0,)), ((), ())),
                                  preferred_element_type=jnp.float32)
            comm_ref[0, 0:DH, g * G * SQ:(g + 1) * G * SQ] = ogt
            comm_ref[0, DH:DH + 1, g * G * SQ:(g + 1) * G * SQ] = lsum
        comm_ref[0, DH + 1:CROWS, :] = jnp.zeros(
            (CROWS - DH - 1, COLS), jnp.float32)

        for h in range(N_DEV - 1):
            rdma = pltpu.make_async_remote_copy(
                src_ref=comm_ref.at[h],
                dst_ref=comm_ref.at[h + 1],
                send_sem=send_sems.at[h],
                recv_sem=recv_sems.at[h],
                device_id=(right,),
                device_id_type=pl.DeviceIdType.MESH,
            )
            rdma.start()
            rdma.wait()

        acc = (comm_ref[0] + comm_ref[1]) + (comm_ref[2] + comm_ref[3])
        outn = acc[0:DH, :] / acc[DH:DH + 1, :]
        attn_t = jnp.concatenate(
            [outn[:, h * SQ:(h + 1) * SQ] for h in range(HQ)], axis=0)
        attn = jnp.transpose(attn_t.astype(jnp.bfloat16))
        out_ref[...] = jnp.dot(attn, wo_ref[...].astype(jnp.bfloat16),
                               preferred_element_type=jnp.float32)

    out = pl.pallas_call(
        body,
        out_shape=jax.ShapeDtypeStruct((SQ, HQ * DH), jnp.float32),
        in_specs=[pl.BlockSpec(memory_space=pltpu.VMEM)] * 5,
        out_specs=pl.BlockSpec(memory_space=pltpu.VMEM),
        scratch_shapes=[
            pltpu.VMEM((N_DEV, CROWS, COLS), jnp.float32),
            pltpu.SemaphoreType.DMA((N_DEV - 1,)),
            pltpu.SemaphoreType.DMA((N_DEV - 1,)),
        ],
        compiler_params=pltpu.CompilerParams(collective_id=0),
    )(x2, Wq, Wo, K2, V2)
    return out.reshape(1, SQ, HQ * DH)


# device time: 29955 ns/iter; 2.0034x vs baseline; 2.0034x over previous
import jax
import jax.numpy as jnp
from jax import lax
from jax.experimental import pallas as pl
from jax.experimental.pallas import tpu as pltpu

N_DEV = 4
SQ = 256
DH = 128
HQ = 8
HKV = 2
G = HQ // HKV
SCALE = 0.08838834764831843
GCOLS = G * SQ
CROWS = DH + 8

_MINE, _FL1, _FR1, _FL2 = 0, 1, 2, 3


def kernel(x, Wq, Wo, K_ext, V_ext):
    skv = K_ext.shape[1]
    x2 = x.reshape(SQ, HQ * DH)
    K2 = K_ext.reshape(skv, HKV * DH)
    V2 = V_ext.reshape(skv, HKV * DH)

    def body(x_ref, wq_ref, wo_ref, k_ref, v_ref, out_ref,
             comm_ref, send_sems, recv_sems):
        my = lax.axis_index("i")
        left = lax.rem(my + N_DEV - 1, N_DEV)
        right = lax.rem(my + 1, N_DEV)

        barrier_sem = pltpu.get_barrier_semaphore()
        for nbr in (left, right):
            pl.semaphore_signal(
                barrier_sem, inc=1,
                device_id=(nbr,), device_id_type=pl.DeviceIdType.MESH,
            )
        pl.semaphore_wait(barrier_sem, 2)

        def desc(src_slot, dst_slot, sem, target):
            return pltpu.make_async_remote_copy(
                src_ref=comm_ref.at[src_slot],
                dst_ref=comm_ref.at[dst_slot],
                send_sem=send_sems.at[sem],
                recv_sem=recv_sems.at[sem],
                device_id=(target,),
                device_id_type=pl.DeviceIdType.MESH,
            )

        dA = [desc(_MINE * HKV + g, _FL1 * HKV + g, g, right) for g in range(HKV)]
        dB = [desc(_MINE * HKV + g, _FR1 * HKV + g, HKV + g, left) for g in range(HKV)]
        dC = [desc(_FL1 * HKV + g, _FL2 * HKV + g, 2 * HKV + g, right) for g in range(HKV)]

        xb = x_ref[...].astype(jnp.bfloat16)
        wqb = wq_ref[...].astype(jnp.bfloat16)
        q = jnp.dot(xb, wqb, preferred_element_type=jnp.float32) * SCALE
        qb = q.astype(jnp.bfloat16)

        for g in range(HKV):
            qg = jnp.concatenate(
                [qb[:, (g * G + j) * DH:(g * G + j + 1) * DH] for j in range(G)],
                axis=0,
            )
            kg = k_ref[:, g * DH:(g + 1) * DH].astype(jnp.bfloat16)
            vg = v_ref[:, g * DH:(g + 1) * DH].astype(jnp.bfloat16)
            st = lax.dot_general(kg, qg, (((1,), (1,)), ((), ())),
                                 preferred_element_type=jnp.float32)
            pt = jnp.exp(st)
            lsum = jnp.sum(pt, axis=0, keepdims=True)
            ogt = lax.dot_general(vg, pt.astype(jnp.bfloat16),
                                  (((0,), (0,)), ((), ())),
                                  preferred_element_type=jnp.float32)
            comm_ref[_MINE * HKV + g, 0:DH, :] = ogt.astype(jnp.bfloat16)
            comm_ref[_MINE * HKV + g, DH:DH + 1, :] = lsum.astype(jnp.bfloat16)
            dA[g].start()
            dB[g].start()

        acc = [None, None]
        for g in range(HKV):
            dA[g].wait_recv()
            dC[g].start()
            acc[g] = (comm_ref[_MINE * HKV + g].astype(jnp.float32)
                      + comm_ref[_FL1 * HKV + g].astype(jnp.float32))
        for g in range(HKV):
            dB[g].wait_recv()
            acc[g] = acc[g] + comm_ref[_FR1 * HKV + g].astype(jnp.float32)
        for g in range(HKV):
            dC[g].wait_recv()
            acc[g] = acc[g] + comm_ref[_FL2 * HKV + g].astype(jnp.float32)

        attn_t = jnp.concatenate(
            [(acc[g][0:DH, j * SQ:(j + 1) * SQ]
              / acc[g][DH:DH + 1, j * SQ:(j + 1) * SQ]).astype(jnp.bfloat16)
             for g in range(HKV) for j in range(G)],
            axis=0,
        )
        attn = jnp.transpose(attn_t)
        out_ref[...] = jnp.dot(attn, wo_ref[...].astype(jnp.bfloat16),
                               preferred_element_type=jnp.float32)

        for d in dA + dB + dC:
            d.wait_send()

    out = pl.pallas_call(
        body,
        out_shape=jax.ShapeDtypeStruct((SQ, HQ * DH), jnp.float32),
        in_specs=[pl.BlockSpec(memory_space=pltpu.VMEM)] * 5,
        out_specs=pl.BlockSpec(memory_space=pltpu.VMEM),
        scratch_shapes=[
            pltpu.VMEM((4 * HKV, CROWS, GCOLS), jnp.bfloat16),
            pltpu.SemaphoreType.DMA((3 * HKV,)),
            pltpu.SemaphoreType.DMA((3 * HKV,)),
        ],
        compiler_params=pltpu.CompilerParams(collective_id=0),
    )(x2, Wq, Wo, K2, V2)
    return out.reshape(1, SQ, HQ * DH)
